# baseline (device time: 40112 ns/iter reference)
import jax
import jax.numpy as jnp
from jax import lax
from jax.experimental import pallas as pl
from jax.experimental.pallas import tpu as pltpu

M = 2048
D = 2048
Q = M // 4
C = 256
NQ = Q // C
NH = NQ // 2

SCALE = 25.6
WIRE_DTYPE = jnp.int8


def kernel(partial, gamma):
    x = partial.reshape(2 * M, D)

    def body(
        x_hbm,
        gamma_ref,
        out_hbm,
        stage,
        send_buf,
        recv_y,
        recv_x,
        recv_z,
        recv_xd,
        recv_zd,
        local_buf,
        out_buf,
        stage_sem,
        local_sem,
        out_sem,
        ysend_sem,
        yrecv_sem,
        xfsend_sem,
        xfrecv_sem,
        zfsend_sem,
        zfrecv_sem,
        xrsend_sem,
        xrrecv_sem,
        zrsend_sem,
        zrrecv_sem,
    ):
        my_x = lax.axis_index("x")
        my_y = lax.axis_index("y")
        my_z = lax.axis_index("z")
        zq = lax.rem(my_z, 2)
        zpz = my_z + 1 - 2 * zq
        ypeer = (my_x, 1 - my_y, my_z)
        xpeer = (1 - my_x, my_y, my_z)
        zpeer = (my_x, my_y, zpz)

        blk0 = my_y * M
        qf0 = (2 * my_x + zq) * Q
        qx0 = (2 * (1 - my_x) + zq) * Q
        qz0 = (2 * my_x + (1 - zq)) * Q
        qd0 = (2 * (1 - my_x) + (1 - zq)) * Q
        send0 = (1 - my_y) * M + qf0

        rows_seq = [qf0 + k * C for k in range(NQ)]
        recv_seq = [(recv_y, k) for k in range(NQ)]
        for k in range(NQ):
            rows_seq += [qx0 + k * C, qz0 + k * C]
            recv_seq += [(recv_x, k), (recv_z, k)]
        rows_seq += [qd0 + k * C for k in range(NQ)]
        recv_seq += [(recv_zd, k) for k in range(NH)]
        recv_seq += [(recv_xd, k) for k in range(NH)]
        NTOT = len(rows_seq)

        def stage_cp(k, slot):
            return pltpu.make_async_copy(
                x_hbm.at[pl.ds(send0 + k * C, C), :],
                stage.at[slot],
                stage_sem.at[slot],
            )

        def local_cp(j, slot):
            return pltpu.make_async_copy(
                x_hbm.at[pl.ds(blk0 + rows_seq[j], C), :],
                local_buf.at[slot],
                local_sem.at[slot],
            )

        stage_cp(0, 0).start()
        stage_cp(1, 1).start()
        local_cp(0, 0).start()
        local_cp(1, 1).start()

        barrier = pltpu.get_barrier_semaphore()
        for p in (ypeer, xpeer, zpeer):
            pl.semaphore_signal(
                barrier, inc=1, device_id=p, device_id_type=pl.DeviceIdType.MESH
            )
        pl.semaphore_wait(barrier, 3)

        rdma_y = []
        for k in range(NQ):
            stage_cp(k, k % 2).wait()
            send_buf[k] = jnp.clip(
                jnp.round(stage[k % 2] * SCALE), -127.0, 127.0
            ).astype(WIRE_DTYPE)
            r = pltpu.make_async_remote_copy(
                src_ref=send_buf.at[k],
                dst_ref=recv_y.at[k],
                send_sem=ysend_sem.at[k],
                recv_sem=yrecv_sem.at[k],
                device_id=ypeer,
                device_id_type=pl.DeviceIdType.MESH,
            )
            r.start()
            rdma_y.append(r)
            if k + 2 < NQ:
                stage_cp(k + 2, k % 2).start()

        out_cps = [None, None]

        def compute(j):
            slot = j % 2
            local_cp(j, slot).wait()
            ref, k = recv_seq[j]
            yv = local_buf[slot] + ref[k].astype(jnp.float32) * (1.0 / SCALE)
            rms = jnp.sqrt(jnp.mean(yv * yv, axis=-1, keepdims=True) + 1e-6)
            if out_cps[slot] is not None:
                out_cps[slot].wait()
            out_buf[slot] = yv / rms * gamma_ref[...][None, :]
            cp = pltpu.make_async_copy(
                out_buf.at[slot],
                out_hbm.at[pl.ds(rows_seq[j], C), :],
                out_sem.at[slot],
            )
            cp.start()
            out_cps[slot] = cp
            if j + 2 < NTOT:
                local_cp(j + 2, slot).start()

        fx, fz = [], []
        j = 0
        for k in range(NQ):
            rdma_y[k].wait_recv()
            r = pltpu.make_async_remote_copy(
                src_ref=recv_y.at[k],
                dst_ref=recv_x.at[k],
                send_sem=xfsend_sem.at[k],
                recv_sem=xfrecv_sem.at[k],
                device_id=xpeer,
                device_id_type=pl.DeviceIdType.MESH,
            )
            r.start()
            fx.append(r)
            r = pltpu.make_async_remote_copy(
                src_ref=recv_y.at[k],
                dst_ref=recv_z.at[k],
                send_sem=zfsend_sem.at[k],
                recv_sem=zfrecv_sem.at[k],
                device_id=zpeer,
                device_id_type=pl.DeviceIdType.MESH,
            )
            r.start()
            fz.append(r)
            compute(j)
            j += 1

        xr, zr = [], []
        for k in range(NQ):
            fx[k].wait_recv()
            if k >= NH:
                r = pltpu.make_async_remote_copy(
                    src_ref=recv_x.at[k],
                    dst_ref=recv_zd.at[k - NH],
                    send_sem=zrsend_sem.at[k - NH],
                    recv_sem=zrrecv_sem.at[k - NH],
                    device_id=zpeer,
                    device_id_type=pl.DeviceIdType.MESH,
                )
                r.start()
                zr.append(r)
            compute(j)
            j += 1
            fz[k].wait_recv()
            if k < NH:
                r = pltpu.make_async_remote_copy(
                    src_ref=recv_z.at[k],
                    dst_ref=recv_xd.at[k],
                    send_sem=xrsend_sem.at[k],
                    recv_sem=xrrecv_sem.at[k],
                    device_id=xpeer,
                    device_id_type=pl.DeviceIdType.MESH,
                )
                r.start()
                xr.append(r)
            compute(j)
            j += 1

        for k in range(NH):
            xr[k].wait_recv()
            compute(j)
            j += 1
        for k in range(NH):
            zr[k].wait_recv()
            compute(j)
            j += 1

        for k in range(NQ):
            rdma_y[k].wait_send()
            fx[k].wait_send()
            fz[k].wait_send()
        for k in range(NH):
            xr[k].wait_send()
            zr[k].wait_send()
        out_cps[0].wait()
        out_cps[1].wait()

    return pl.pallas_call(
        body,
        out_shape=jax.ShapeDtypeStruct((M, D), jnp.float32),
        in_specs=[
            pl.BlockSpec(memory_space=pl.ANY),
            pl.BlockSpec(memory_space=pltpu.VMEM),
        ],
        out_specs=pl.BlockSpec(memory_space=pl.ANY),
        scratch_shapes=[
            pltpu.VMEM((2, C, D), jnp.float32),
            pltpu.VMEM((NQ, C, D), WIRE_DTYPE),
            pltpu.VMEM((NQ, C, D), WIRE_DTYPE),
            pltpu.VMEM((NQ, C, D), WIRE_DTYPE),
            pltpu.VMEM((NQ, C, D), WIRE_DTYPE),
            pltpu.VMEM((NH, C, D), WIRE_DTYPE),
            pltpu.VMEM((NH, C, D), WIRE_DTYPE),
            pltpu.VMEM((2, C, D), jnp.float32),
            pltpu.VMEM((2, C, D), jnp.float32),
            pltpu.SemaphoreType.DMA((2,)),
            pltpu.SemaphoreType.DMA((2,)),
            pltpu.SemaphoreType.DMA((2,)),
            pltpu.SemaphoreType.DMA((NQ,)),
            pltpu.SemaphoreType.DMA((NQ,)),
            pltpu.SemaphoreType.DMA((NQ,)),
            pltpu.SemaphoreType.DMA((NQ,)),
            pltpu.SemaphoreType.DMA((NQ,)),
            pltpu.SemaphoreType.DMA((NQ,)),
            pltpu.SemaphoreType.DMA((NH,)),
            pltpu.SemaphoreType.DMA((NH,)),
            pltpu.SemaphoreType.DMA((NH,)),
            pltpu.SemaphoreType.DMA((NH,)),
        ],
        compiler_params=pltpu.CompilerParams(collective_id=0),
    )(x, gamma)


# device time: 34598 ns/iter; 1.1594x vs baseline; 1.1594x over previous
import jax
import jax.numpy as jnp
from jax import lax
from jax.experimental import pallas as pl
from jax.experimental.pallas import tpu as pltpu

M = 2048
D = 2048
Q = M // 4
C = 128
NQ = Q // C
NY = 2 * NQ

SCALE = 25.6
WIRE_DTYPE = jnp.int8


def kernel(partial, gamma):
    x = partial.reshape(2 * M, D)

    def body(
        x_hbm,
        gamma_ref,
        out_hbm,
        stage,
        send_buf,
        recv_y,
        recv_x,
        recv_z,
        local_buf,
        out_buf,
        stage_sem,
        local_sem,
        out_sem,
        ysend_sem,
        yrecv_sem,
        xfsend_sem,
        xfrecv_sem,
        zfsend_sem,
        zfrecv_sem,
    ):
        my_x = lax.axis_index("x")
        my_y = lax.axis_index("y")
        my_z = lax.axis_index("z")
        zq = lax.rem(my_z, 2)
        zpz = my_z + 1 - 2 * zq
        ypeer = (my_x, 1 - my_y, my_z)
        xpeer = (1 - my_x, my_y, my_z)
        zpeer = (my_x, my_y, zpz)

        blk0 = my_y * M
        qf0 = (2 * my_x + zq) * Q
        qd0 = (2 * (1 - my_x) + (1 - zq)) * Q
        qx0 = (2 * (1 - my_x) + zq) * Q
        qz0 = (2 * my_x + (1 - zq)) * Q
        peer_blk0 = (1 - my_y) * M

        def send_row(i):
            if i < NQ:
                return peer_blk0 + qf0 + i * C
            return peer_blk0 + qd0 + (i - NQ) * C

        rows_seq = [qf0 + k * C for k in range(NQ)]
        recv_seq = [(recv_y, k) for k in range(NQ)]
        for k in range(NQ):
            rows_seq += [qx0 + k * C, qz0 + k * C, qd0 + k * C]
            recv_seq += [(recv_x, k), (recv_z, k), (recv_y, NQ + k)]
        NTOT = len(rows_seq)

        def stage_cp(i, slot):
            return pltpu.make_async_copy(
                x_hbm.at[pl.ds(send_row(i), C), :],
                stage.at[slot],
                stage_sem.at[slot],
            )

        def local_cp(j, slot):
            return pltpu.make_async_copy(
                x_hbm.at[pl.ds(blk0 + rows_seq[j], C), :],
                local_buf.at[slot],
                local_sem.at[slot],
            )

        stage_cp(0, 0).start()
        stage_cp(1, 1).start()
        local_cp(0, 0).start()
        local_cp(1, 1).start()

        barrier = pltpu.get_barrier_semaphore()
        for p in (ypeer, xpeer, zpeer):
            pl.semaphore_signal(
                barrier, inc=1, device_id=p, device_id_type=pl.DeviceIdType.MESH
            )
        pl.semaphore_wait(barrier, 3)

        rdma_y = []
        for i in range(NY):
            stage_cp(i, i % 2).wait()
            send_buf[i] = jnp.clip(
                jnp.round(stage[i % 2] * SCALE), -127.0, 127.0
            ).astype(WIRE_DTYPE)
            r = pltpu.make_async_remote_copy(
                src_ref=send_buf.at[i],
                dst_ref=recv_y.at[i],
                send_sem=ysend_sem.at[i],
                recv_sem=yrecv_sem.at[i],
                device_id=ypeer,
                device_id_type=pl.DeviceIdType.MESH,
            )
            r.start()
            rdma_y.append(r)
            if i + 2 < NY:
                stage_cp(i + 2, i % 2).start()

        out_cps = [None, None]

        def compute(j):
            slot = j % 2
            local_cp(j, slot).wait()
            ref, k = recv_seq[j]
            yv = local_buf[slot] + ref[k].astype(jnp.float32) * (1.0 / SCALE)
            rms = jnp.sqrt(jnp.mean(yv * yv, axis=-1, keepdims=True) + 1e-6)
            if out_cps[slot] is not None:
                out_cps[slot].wait()
            out_buf[slot] = yv / rms * gamma_ref[...][None, :]
            cp = pltpu.make_async_copy(
                out_buf.at[slot],
                out_hbm.at[pl.ds(rows_seq[j], C), :],
                out_sem.at[slot],
            )
            cp.start()
            out_cps[slot] = cp
            if j + 2 < NTOT:
                local_cp(j + 2, slot).start()

        fx, fz = [], []
        j = 0
        for k in range(NQ):
            rdma_y[k].wait_recv()
            r = pltpu.make_async_remote_copy(
                src_ref=recv_y.at[k],
                dst_ref=recv_x.at[k],
                send_sem=xfsend_sem.at[k],
                recv_sem=xfrecv_sem.at[k],
                device_id=xpeer,
                device_id_type=pl.DeviceIdType.MESH,
            )
            r.start()
            fx.append(r)
            r = pltpu.make_async_remote_copy(
                src_ref=recv_y.at[k],
                dst_ref=recv_z.at[k],
                send_sem=zfsend_sem.at[k],
                recv_sem=zfrecv_sem.at[k],
                device_id=zpeer,
                device_id_type=pl.DeviceIdType.MESH,
            )
            r.start()
            fz.append(r)
            compute(j)
            j += 1

        for k in range(NQ):
            fx[k].wait_recv()
            compute(j)
            j += 1
            fz[k].wait_recv()
            compute(j)
            j += 1
            rdma_y[NQ + k].wait_recv()
            compute(j)
            j += 1

        for i in range(NY):
            rdma_y[i].wait_send()
        for k in range(NQ):
            fx[k].wait_send()
            fz[k].wait_send()
        out_cps[0].wait()
        out_cps[1].wait()

    return pl.pallas_call(
        body,
        out_shape=jax.ShapeDtypeStruct((M, D), jnp.float32),
        in_specs=[
            pl.BlockSpec(memory_space=pl.ANY),
            pl.BlockSpec(memory_space=pltpu.VMEM),
        ],
        out_specs=pl.BlockSpec(memory_space=pl.ANY),
        scratch_shapes=[
            pltpu.VMEM((2, C, D), jnp.float32),
            pltpu.VMEM((NY, C, D), WIRE_DTYPE),
            pltpu.VMEM((NY, C, D), WIRE_DTYPE),
            pltpu.VMEM((NQ, C, D), WIRE_DTYPE),
            pltpu.VMEM((NQ, C, D), WIRE_DTYPE),
            pltpu.VMEM((2, C, D), jnp.float32),
            pltpu.VMEM((2, C, D), jnp.float32),
            pltpu.SemaphoreType.DMA((2,)),
            pltpu.SemaphoreType.DMA((2,)),
            pltpu.SemaphoreType.DMA((2,)),
            pltpu.SemaphoreType.DMA((NY,)),
            pltpu.SemaphoreType.DMA((NY,)),
            pltpu.SemaphoreType.DMA((NQ,)),
            pltpu.SemaphoreType.DMA((NQ,)),
            pltpu.SemaphoreType.DMA((NQ,)),
            pltpu.SemaphoreType.DMA((NQ,)),
        ],
        compiler_params=pltpu.CompilerParams(collective_id=0),
    )(x, gamma)
